# baseline (speedup 1.0000x reference)
import functools

import jax
import jax.numpy as jnp
from jax import lax
from jax.experimental import pallas as pl
from jax.experimental.pallas import tpu as pltpu
from jax.experimental.pallas import tpu_sc as plsc

N = 10000
E = 320000
DIN = 128
DH = 128
R = 8
L = 16
NC = 2
NS = 16
NW = NC * NS
EPW = E // NW
EPS = E // NS
CP = 2000
RN = R * N
CNT_ROWS = 5120
CH = 400
CB = 400
NB = CH // CB
ROWS_PER_SUB = N // NS

_mesh = plsc.VectorSubcoreMesh(core_axis_name="c", subcore_axis_name="s")

_i32 = jnp.int32
_f32 = jnp.float32


def _zeros16f():
    return jnp.zeros((L,), _f32)


def _iota16():
    return lax.iota(_i32, L)


_GDN = lax.GatherDimensionNumbers(offset_dims=(), collapsed_slice_dims=(0,),
                                  start_index_map=(0,))


def _bcast_lane(v, i):
    idx = jnp.full((L, 1), i, dtype=_i32)
    return lax.gather(v, idx, _GDN, (1,),
                      mode=lax.GatherScatterMode.PROMISE_IN_BOUNDS)


@functools.partial(
    pl.kernel,
    out_type=(
        jax.ShapeDtypeStruct((E,), _i32),
        jax.ShapeDtypeStruct((E,), _f32),
    ),
    mesh=_mesh,
    compiler_params=pltpu.CompilerParams(needs_layout_passes=False),
    scratch_types=[
        pltpu.VMEM((CNT_ROWS * L,), _f32),
        pltpu.VMEM((CP,), _i32),
        pltpu.VMEM((CP,), _i32),
        pltpu.VMEM((CP,), _i32),
        pltpu.VMEM((CP,), _f32),
        pltpu.VMEM((CNT_ROWS * L // (NS * 8),), _f32),
        pltpu.VMEM((CNT_ROWS * L // (NS * 8),), _f32),
        pltpu.VMEM_SHARED((NS, CNT_ROWS * L // 8), _f32),
        pltpu.VMEM_SHARED((CNT_ROWS * L,), _f32),
    ],
)
def _sc_prep(src_hbm, dst_hbm, et_hbm, gidx_hbm, norm_hbm,
             cnt_v, ia_v, ib_v, ic_v, fc_v, tmp_v, acc_v,
             shared_all, shared_comb):
    cid = lax.axis_index("c")
    sid = lax.axis_index("s")
    wid = sid * NC + cid

    @plsc.parallel_loop(0, CNT_ROWS, step=1, unroll=8)
    def _zero_row(i):
        cnt_v[pl.ds(i * L, L)] = _zeros16f()

    for ch in range(EPW // CP):
        base = wid * EPW + ch * CP
        pltpu.sync_copy(src_hbm.at[pl.ds(base, CP)], ia_v)
        pltpu.sync_copy(et_hbm.at[pl.ds(base, CP)], ib_v)

        @plsc.parallel_loop(0, CP // L, step=1, unroll=4)
        def _flat_combine(j):
            sj = pl.ds(j * L, L)
            ic_v[sj] = ib_v[sj] * N + ia_v[sj]

        pltpu.sync_copy(ic_v, gidx_hbm.at[pl.ds(base, CP)])

    ones16 = jnp.ones((L,), _f32)

    def _count_group(j, _):
        s = pl.ds(j * L, L)
        sidx = ib_v[s] * N + ia_v[s]
        plsc.addupdate_scatter(cnt_v, [sidx], ones16)
        return _

    for ch in range(EPS // CP):
        base = sid * EPS + ch * CP
        pltpu.sync_copy(dst_hbm.at[pl.ds(base, CP)], ia_v)
        pltpu.sync_copy(et_hbm.at[pl.ds(base, CP)], ib_v)
        lax.fori_loop(0, CP // L, _count_group, None)

    npass = 8
    segp = CNT_ROWS * L // npass
    seg = segp // NS

    for p in range(npass):
        pltpu.sync_copy(cnt_v.at[pl.ds(p * segp, segp)], shared_all.at[sid])
        plsc.subcore_barrier()
        pltpu.sync_copy(shared_all.at[0, pl.ds(sid * seg, seg)], acc_v)
        for t in range(1, NS):
            pltpu.sync_copy(shared_all.at[t, pl.ds(sid * seg, seg)], tmp_v)

            @plsc.parallel_loop(0, seg // L, step=1, unroll=4)
            def _acc_group(j):
                sj = pl.ds(j * L, L)
                acc_v[sj] = acc_v[sj] + tmp_v[sj]
        pltpu.sync_copy(acc_v,
                        shared_comb.at[pl.ds(p * segp + sid * seg, seg)])
        plsc.subcore_barrier()
    pltpu.sync_copy(shared_comb, cnt_v)

    for ch in range(EPW // CP):
        base = wid * EPW + ch * CP
        pltpu.sync_copy(dst_hbm.at[pl.ds(base, CP)], ia_v)
        pltpu.sync_copy(et_hbm.at[pl.ds(base, CP)], ib_v)

        @plsc.parallel_loop(0, CP // L, step=1, unroll=4)
        def _norm_chunk(j):
            sj = pl.ds(j * L, L)
            sidx = ib_v[sj] * N + ia_v[sj]
            cv = plsc.load_gather(cnt_v, [sidx])
            fc_v[sj] = 1.0 / cv

        pltpu.sync_copy(fc_v, norm_hbm.at[pl.ds(base, CP)])


DF = DH // NC


NCHUNK = EPS // CH


@functools.partial(
    pl.kernel,
    out_type=jax.ShapeDtypeStruct((NC * N, DF), _f32),
    mesh=_mesh,
    compiler_params=pltpu.CompilerParams(needs_layout_passes=False,
                                         use_tc_tiling_on_sc=False),
    scratch_types=[
        pltpu.VMEM((CH,), _i32),
        pltpu.VMEM((CH,), _i32),
        pltpu.VMEM((CH,), _i32),
        pltpu.VMEM((CH,), _i32),
        pltpu.VMEM((CH,), _f32),
        pltpu.VMEM((CH,), _f32),
        pltpu.VMEM((CH, DF), _f32),
        pltpu.VMEM((CH, DF), _f32),
        pltpu.VMEM_SHARED((N, DF), _f32),
        pltpu.SemaphoreType.DMA,
        pltpu.SemaphoreType.DMA,
        pltpu.SemaphoreType.DMA,
        pltpu.SemaphoreType.DMA,
        pltpu.SemaphoreType.DMA,
        pltpu.SemaphoreType.DMA,
    ],
)
def _sc_agg(hall_hbm, gidx_hbm, dst_hbm, norm_hbm, out_hbm,
            gidx0_v, gidx1_v, dst0_v, dst1_v, norm0_v, norm1_v,
            rows0_v, rows1_v, shared_agg,
            isem0, isem1, gsem0, gsem1, ssem0, ssem1):
    cid = lax.axis_index("c")
    sid = lax.axis_index("s")
    ebase0 = sid * EPS
    hoff = cid * RN

    slot0 = (gidx0_v, dst0_v, norm0_v, rows0_v, isem0, gsem0, ssem0)
    slot1 = (gidx1_v, dst1_v, norm1_v, rows1_v, isem1, gsem1, ssem1)

    zc = 125

    def _zero_row(i, _):
        for j in range(DF // L):
            rows0_v[i, pl.ds(j * L, L)] = _zeros16f()
        return _

    lax.fori_loop(0, zc, _zero_row, None)
    for m in range(ROWS_PER_SUB // zc):
        pltpu.sync_copy(rows0_v.at[pl.ds(0, zc)],
                        shared_agg.at[pl.ds(sid * ROWS_PER_SUB + m * zc, zc)])
    plsc.subcore_barrier()

    def _start_idx(c, slot):
        gv, dv, nv, _, isem, _, _ = slot
        eb = ebase0 + c * CH
        pltpu.async_copy(gidx_hbm.at[pl.ds(eb, CH)], gv, isem)
        pltpu.async_copy(dst_hbm.at[pl.ds(eb, CH)], dv, isem)
        pltpu.async_copy(norm_hbm.at[pl.ds(eb, CH)], nv, isem)

    def _wait_idx(c, slot):
        gv, dv, nv, _, isem, _, _ = slot
        eb = ebase0 + c * CH
        pltpu.make_async_copy(gidx_hbm.at[pl.ds(eb, CH)], gv, isem).wait()
        pltpu.make_async_copy(dst_hbm.at[pl.ds(eb, CH)], dv, isem).wait()
        pltpu.make_async_copy(norm_hbm.at[pl.ds(eb, CH)], nv, isem).wait()

    def _rebase_and_gather(slot):
        gv, _, _, rows, _, gsem, _ = slot
        for j in range(CH // L):
            sl = pl.ds(j * L, L)
            gv[sl] = gv[sl] + hoff
        for k in range(NB):
            pltpu.async_copy(hall_hbm.at[gv.at[pl.ds(k * CB, CB)]],
                             rows.at[pl.ds(k * CB, CB)], gsem)

    def _wait_gather(slot):
        gv, _, _, rows, _, gsem, _ = slot
        for k in range(NB):
            pltpu.make_async_copy(hall_hbm.at[gv.at[pl.ds(k * CB, CB)]],
                                  rows.at[pl.ds(k * CB, CB)], gsem).wait()

    def _scale(slot):
        _, _, nv_ref, rows, _, _, _ = slot

        @plsc.parallel_loop(0, CH // L, step=1, unroll=4)
        def body(g):
            nv = nv_ref[pl.ds(g * L, L)]
            for i in range(L):
                sv = _bcast_lane(nv, i)
                r = g * L + i
                for j in range(DF // L):
                    sl = pl.ds(j * L, L)
                    rows[r, sl] = rows[r, sl] * sv

    def _start_scatter(slot):
        _, dv, _, rows, _, _, ssem = slot
        for k in range(NB):
            pltpu.async_copy(rows.at[pl.ds(k * CB, CB)],
                             shared_agg.at[dv.at[pl.ds(k * CB, CB)]], ssem,
                             add=True)

    def _wait_scatter(slot):
        _, dv, _, rows, _, _, ssem = slot
        for k in range(NB):
            pltpu.make_async_copy(rows.at[pl.ds(k * CB, CB)],
                                  shared_agg.at[dv.at[pl.ds(k * CB, CB)]],
                                  ssem).wait()

    _start_idx(0, slot0)
    _start_idx(1, slot1)

    def _pipe(g, _):
        a = 2 * g
        b = 2 * g + 1
        _wait_idx(a, slot0)
        _rebase_and_gather(slot0)
        _wait_idx(b, slot1)
        _rebase_and_gather(slot1)
        _wait_gather(slot0)
        _scale(slot0)
        _start_scatter(slot0)
        _wait_gather(slot1)
        _scale(slot1)
        _start_scatter(slot1)
        na = jnp.minimum(a + 2, NCHUNK - 2)
        nb = jnp.minimum(b + 2, NCHUNK - 1)
        _wait_scatter(slot0)
        _start_idx(na, slot0)
        _wait_scatter(slot1)
        _start_idx(nb, slot1)
        return _

    lax.fori_loop(0, NCHUNK // 2, _pipe, None)
    _wait_idx(NCHUNK - 2, slot0)
    _wait_idx(NCHUNK - 1, slot1)

    plsc.subcore_barrier()
    pltpu.sync_copy(
        shared_agg.at[pl.ds(sid * ROWS_PER_SUB, ROWS_PER_SUB)],
        out_hbm.at[pl.ds(cid * N + sid * ROWS_PER_SUB, ROWS_PER_SUB)])


def _tc_pre_body(x_ref, w_ref, o_ref):
    h = jnp.dot(x_ref[...], w_ref[0], preferred_element_type=_f32)
    o_ref[0, 0] = h[:, :DF]
    o_ref[1, 0] = h[:, DF:]


def _tc_pre(x, w_rel):
    return pl.pallas_call(
        _tc_pre_body,
        grid=(R,),
        in_specs=[
            pl.BlockSpec((N, DIN), lambda r: (0, 0)),
            pl.BlockSpec((1, DIN, DH), lambda r: (r, 0, 0)),
        ],
        out_specs=pl.BlockSpec((NC, 1, N, DF), lambda r: (0, r, 0, 0)),
        out_shape=jax.ShapeDtypeStruct((NC, R, N, DF), _f32),
    )(x, w_rel)


def _tc_post_body(aggp_ref, x_ref, w_ref, pv_ref, o_ref):
    agg = jnp.concatenate([aggp_ref[:N], aggp_ref[N:]], axis=1)
    z = (agg
         + jnp.dot(x_ref[...], w_ref[...], preferred_element_type=_f32)
         + pv_ref[0])
    m = jnp.mean(z, axis=0)
    d = z - m
    v = jnp.mean(d * d, axis=0)
    zn = d * lax.rsqrt(v + 1e-5) * pv_ref[1] + pv_ref[2]
    o_ref[...] = jnp.where(zn > 0, zn, pv_ref[3] * zn)


def _tc_post(aggp, x, w_root, pv):
    return pl.pallas_call(
        _tc_post_body,
        out_shape=jax.ShapeDtypeStruct((N, DH), _f32),
    )(aggp, x, w_root, pv)


def _tc_postpre_body(aggp_ref, x_ref, w_ref, pv_ref, wrel_ref, h_ref, o_ref):
    r = pl.program_id(0)

    @pl.when(r == 0)
    def _():
        _tc_post_body(aggp_ref, x_ref, w_ref, pv_ref, h_ref)

    hh = jnp.dot(h_ref[...], wrel_ref[0], preferred_element_type=_f32)
    o_ref[0, 0] = hh[:, :DF]
    o_ref[1, 0] = hh[:, DF:]


def _tc_postpre(aggp, x, w_root, pv, w_rel):
    return pl.pallas_call(
        _tc_postpre_body,
        grid=(R,),
        in_specs=[
            pl.BlockSpec((NC * N, DF), lambda r: (0, 0)),
            pl.BlockSpec((N, DIN), lambda r: (0, 0)),
            pl.BlockSpec((DIN, DH), lambda r: (0, 0)),
            pl.BlockSpec((4, DH), lambda r: (0, 0)),
            pl.BlockSpec((1, DIN, DH), lambda r: (r, 0, 0)),
        ],
        out_specs=[
            pl.BlockSpec((N, DH), lambda r: (0, 0)),
            pl.BlockSpec((NC, 1, N, DF), lambda r: (0, r, 0, 0)),
        ],
        out_shape=[
            jax.ShapeDtypeStruct((N, DH), _f32),
            jax.ShapeDtypeStruct((NC, R, N, DF), _f32),
        ],
    )(aggp, x, w_root, pv, w_rel)


def kernel(x, edge_index, edge_attr, params):
    src = edge_index[0]
    dst = edge_index[1]
    et = edge_attr[:, 4].astype(_i32)

    gidx, norm = _sc_prep(src, dst, et)

    hall = _tc_pre(x, params["W_rel1"])

    def pv(i):
        return jnp.stack([params[f"b{i}"], params[f"g{i}"],
                          params[f"be{i}"], params[f"a{i}"]])

    aggp = _sc_agg(hall.reshape(NC * RN, DF), gidx, dst, norm)
    h1, hall = _tc_postpre(aggp, x, params["W_root1"], pv("1"),
                           params["W_rel2"])
    aggp = _sc_agg(hall.reshape(NC * RN, DF), gidx, dst, norm)
    h2, hall = _tc_postpre(aggp, h1, params["W_root2"], pv("2"),
                           params["W_rel3"])
    aggp = _sc_agg(hall.reshape(NC * RN, DF), gidx, dst, norm)
    return _tc_post(aggp, h2, params["W_root3"], pv("3"))


# speedup vs baseline: 1.0029x; 1.0029x over previous
import functools

import jax
import jax.numpy as jnp
from jax import lax
from jax.experimental import pallas as pl
from jax.experimental.pallas import tpu as pltpu
from jax.experimental.pallas import tpu_sc as plsc

N = 10000
E = 320000
DIN = 128
DH = 128
R = 8
L = 16
NC = 2
NS = 16
NW = NC * NS
EPW = E // NW
EPS = E // NS
CP = 2000
RN = R * N
CNT_ROWS = 5120
CH = 400
CB = 400
NB = CH // CB
ROWS_PER_SUB = N // NS

_mesh = plsc.VectorSubcoreMesh(core_axis_name="c", subcore_axis_name="s")

_i32 = jnp.int32
_f32 = jnp.float32


def _zeros16f():
    return jnp.zeros((L,), _f32)


def _iota16():
    return lax.iota(_i32, L)


_GDN = lax.GatherDimensionNumbers(offset_dims=(), collapsed_slice_dims=(0,),
                                  start_index_map=(0,))


def _bcast_lane(v, i):
    idx = jnp.full((L, 1), i, dtype=_i32)
    return lax.gather(v, idx, _GDN, (1,),
                      mode=lax.GatherScatterMode.PROMISE_IN_BOUNDS)


@functools.partial(
    pl.kernel,
    out_type=(
        jax.ShapeDtypeStruct((E,), _i32),
        jax.ShapeDtypeStruct((E,), _f32),
    ),
    mesh=_mesh,
    compiler_params=pltpu.CompilerParams(needs_layout_passes=False),
    scratch_types=[
        pltpu.VMEM((CNT_ROWS * L,), _f32),
        pltpu.VMEM((CP,), _i32),
        pltpu.VMEM((CP,), _i32),
        pltpu.VMEM((CP,), _i32),
        pltpu.VMEM((CP,), _f32),
        pltpu.VMEM((CNT_ROWS * L // (NS * 8),), _f32),
        pltpu.VMEM((CNT_ROWS * L // (NS * 8),), _f32),
        pltpu.VMEM_SHARED((NS, CNT_ROWS * L // 8), _f32),
        pltpu.VMEM_SHARED((CNT_ROWS * L,), _f32),
    ],
)
def _sc_prep(src_hbm, dst_hbm, et_hbm, gidx_hbm, norm_hbm,
             cnt_v, ia_v, ib_v, ic_v, fc_v, tmp_v, acc_v,
             shared_all, shared_comb):
    cid = lax.axis_index("c")
    sid = lax.axis_index("s")
    wid = sid * NC + cid

    @plsc.parallel_loop(0, CNT_ROWS, step=1, unroll=8)
    def _zero_row(i):
        cnt_v[pl.ds(i * L, L)] = _zeros16f()

    for ch in range(EPW // CP):
        base = wid * EPW + ch * CP
        pltpu.sync_copy(src_hbm.at[pl.ds(base, CP)], ia_v)
        pltpu.sync_copy(et_hbm.at[pl.ds(base, CP)], ib_v)

        @plsc.parallel_loop(0, CP // L, step=1, unroll=4)
        def _flat_combine(j):
            sj = pl.ds(j * L, L)
            ic_v[sj] = ib_v[sj] * N + ia_v[sj]

        pltpu.sync_copy(ic_v, gidx_hbm.at[pl.ds(base, CP)])

    ones16 = jnp.ones((L,), _f32)

    def _count_group(j, _):
        s = pl.ds(j * L, L)
        sidx = ib_v[s] * N + ia_v[s]
        plsc.addupdate_scatter(cnt_v, [sidx], ones16)
        return _

    for ch in range(EPS // CP):
        base = sid * EPS + ch * CP
        pltpu.sync_copy(dst_hbm.at[pl.ds(base, CP)], ia_v)
        pltpu.sync_copy(et_hbm.at[pl.ds(base, CP)], ib_v)
        lax.fori_loop(0, CP // L, _count_group, None)

    npass = 8
    segp = CNT_ROWS * L // npass
    seg = segp // NS

    for p in range(npass):
        pltpu.sync_copy(cnt_v.at[pl.ds(p * segp, segp)], shared_all.at[sid])
        plsc.subcore_barrier()
        pltpu.sync_copy(shared_all.at[0, pl.ds(sid * seg, seg)], acc_v)
        for t in range(1, NS):
            pltpu.sync_copy(shared_all.at[t, pl.ds(sid * seg, seg)], tmp_v)

            @plsc.parallel_loop(0, seg // L, step=1, unroll=4)
            def _acc_group(j):
                sj = pl.ds(j * L, L)
                acc_v[sj] = acc_v[sj] + tmp_v[sj]
        pltpu.sync_copy(acc_v,
                        shared_comb.at[pl.ds(p * segp + sid * seg, seg)])
        plsc.subcore_barrier()
    pltpu.sync_copy(shared_comb, cnt_v)

    for ch in range(EPW // CP):
        base = wid * EPW + ch * CP
        pltpu.sync_copy(dst_hbm.at[pl.ds(base, CP)], ia_v)
        pltpu.sync_copy(et_hbm.at[pl.ds(base, CP)], ib_v)

        @plsc.parallel_loop(0, CP // L, step=1, unroll=4)
        def _norm_chunk(j):
            sj = pl.ds(j * L, L)
            sidx = ib_v[sj] * N + ia_v[sj]
            cv = plsc.load_gather(cnt_v, [sidx])
            fc_v[sj] = 1.0 / cv

        pltpu.sync_copy(fc_v, norm_hbm.at[pl.ds(base, CP)])


DF = DH // NC


NCHUNK = EPS // CH


@functools.partial(
    pl.kernel,
    out_type=jax.ShapeDtypeStruct((NC * N, DF), _f32),
    mesh=_mesh,
    compiler_params=pltpu.CompilerParams(needs_layout_passes=False,
                                         use_tc_tiling_on_sc=False),
    scratch_types=[
        pltpu.VMEM((CH,), _i32),
        pltpu.VMEM((CH,), _i32),
        pltpu.VMEM((CH,), _i32),
        pltpu.VMEM((CH,), _i32),
        pltpu.VMEM((CH,), _f32),
        pltpu.VMEM((CH,), _f32),
        pltpu.VMEM((CH, DF), _f32),
        pltpu.VMEM((CH, DF), _f32),
        pltpu.VMEM_SHARED((N, DF), _f32),
        pltpu.SemaphoreType.DMA,
        pltpu.SemaphoreType.DMA,
        pltpu.SemaphoreType.DMA,
        pltpu.SemaphoreType.DMA,
        pltpu.SemaphoreType.DMA,
        pltpu.SemaphoreType.DMA,
    ],
)
def _sc_agg(hall_hbm, gidx_hbm, dst_hbm, norm_hbm, out_hbm,
            gidx0_v, gidx1_v, dst0_v, dst1_v, norm0_v, norm1_v,
            rows0_v, rows1_v, shared_agg,
            isem0, isem1, gsem0, gsem1, ssem0, ssem1):
    cid = lax.axis_index("c")
    sid = lax.axis_index("s")
    ebase0 = sid * EPS
    hoff = cid * RN

    slot0 = (gidx0_v, dst0_v, norm0_v, rows0_v, isem0, gsem0, ssem0)
    slot1 = (gidx1_v, dst1_v, norm1_v, rows1_v, isem1, gsem1, ssem1)

    zc = 125

    def _zero_row(i, _):
        for j in range(DF // L):
            rows0_v[i, pl.ds(j * L, L)] = _zeros16f()
        return _

    lax.fori_loop(0, zc, _zero_row, None)
    for m in range(ROWS_PER_SUB // zc):
        pltpu.sync_copy(rows0_v.at[pl.ds(0, zc)],
                        shared_agg.at[pl.ds(sid * ROWS_PER_SUB + m * zc, zc)])
    plsc.subcore_barrier()

    def _start_idx(c, slot):
        gv, dv, nv, _, isem, _, _ = slot
        eb = ebase0 + c * CH
        pltpu.async_copy(gidx_hbm.at[pl.ds(eb, CH)], gv, isem)
        pltpu.async_copy(dst_hbm.at[pl.ds(eb, CH)], dv, isem)
        pltpu.async_copy(norm_hbm.at[pl.ds(eb, CH)], nv, isem)

    def _wait_idx(c, slot):
        gv, dv, nv, _, isem, _, _ = slot
        eb = ebase0 + c * CH
        pltpu.make_async_copy(gidx_hbm.at[pl.ds(eb, CH)], gv, isem).wait()
        pltpu.make_async_copy(dst_hbm.at[pl.ds(eb, CH)], dv, isem).wait()
        pltpu.make_async_copy(norm_hbm.at[pl.ds(eb, CH)], nv, isem).wait()

    def _rebase_and_gather(slot):
        gv, _, _, rows, _, gsem, _ = slot
        for j in range(CH // L):
            sl = pl.ds(j * L, L)
            gv[sl] = gv[sl] + hoff
        for k in range(NB):
            pltpu.async_copy(hall_hbm.at[gv.at[pl.ds(k * CB, CB)]],
                             rows.at[pl.ds(k * CB, CB)], gsem)

    def _wait_gather(slot):
        gv, _, _, rows, _, gsem, _ = slot
        for k in range(NB):
            pltpu.make_async_copy(hall_hbm.at[gv.at[pl.ds(k * CB, CB)]],
                                  rows.at[pl.ds(k * CB, CB)], gsem).wait()

    def _scale(slot):
        _, _, nv_ref, rows, _, _, _ = slot

        @plsc.parallel_loop(0, CH // L, step=1, unroll=2)
        def body(g):
            nv = nv_ref[pl.ds(g * L, L)]
            for i in range(L):
                sv = _bcast_lane(nv, i)
                r = g * L + i
                for j in range(DF // L):
                    sl = pl.ds(j * L, L)
                    rows[r, sl] = rows[r, sl] * sv

    def _start_scatter(slot):
        _, dv, _, rows, _, _, ssem = slot
        for k in range(NB):
            pltpu.async_copy(rows.at[pl.ds(k * CB, CB)],
                             shared_agg.at[dv.at[pl.ds(k * CB, CB)]], ssem,
                             add=True)

    def _wait_scatter(slot):
        _, dv, _, rows, _, _, ssem = slot
        for k in range(NB):
            pltpu.make_async_copy(rows.at[pl.ds(k * CB, CB)],
                                  shared_agg.at[dv.at[pl.ds(k * CB, CB)]],
                                  ssem).wait()

    _start_idx(0, slot0)
    _start_idx(1, slot1)

    def _pipe(g, _):
        a = 2 * g
        b = 2 * g + 1
        _wait_idx(a, slot0)
        _rebase_and_gather(slot0)
        _wait_idx(b, slot1)
        _rebase_and_gather(slot1)
        _wait_gather(slot0)
        _scale(slot0)
        _start_scatter(slot0)
        _wait_gather(slot1)
        _scale(slot1)
        _start_scatter(slot1)
        na = jnp.minimum(a + 2, NCHUNK - 2)
        nb = jnp.minimum(b + 2, NCHUNK - 1)
        _wait_scatter(slot0)
        _start_idx(na, slot0)
        _wait_scatter(slot1)
        _start_idx(nb, slot1)
        return _

    lax.fori_loop(0, NCHUNK // 2, _pipe, None)
    _wait_idx(NCHUNK - 2, slot0)
    _wait_idx(NCHUNK - 1, slot1)

    plsc.subcore_barrier()
    pltpu.sync_copy(
        shared_agg.at[pl.ds(sid * ROWS_PER_SUB, ROWS_PER_SUB)],
        out_hbm.at[pl.ds(cid * N + sid * ROWS_PER_SUB, ROWS_PER_SUB)])


def _tc_pre_body(x_ref, w_ref, o_ref):
    h = jnp.dot(x_ref[...], w_ref[0], preferred_element_type=_f32)
    o_ref[0, 0] = h[:, :DF]
    o_ref[1, 0] = h[:, DF:]


def _tc_pre(x, w_rel):
    return pl.pallas_call(
        _tc_pre_body,
        grid=(R,),
        in_specs=[
            pl.BlockSpec((N, DIN), lambda r: (0, 0)),
            pl.BlockSpec((1, DIN, DH), lambda r: (r, 0, 0)),
        ],
        out_specs=pl.BlockSpec((NC, 1, N, DF), lambda r: (0, r, 0, 0)),
        out_shape=jax.ShapeDtypeStruct((NC, R, N, DF), _f32),
    )(x, w_rel)


def _tc_post_body(aggp_ref, x_ref, w_ref, pv_ref, o_ref):
    agg = jnp.concatenate([aggp_ref[:N], aggp_ref[N:]], axis=1)
    z = (agg
         + jnp.dot(x_ref[...], w_ref[...], preferred_element_type=_f32)
         + pv_ref[0])
    m = jnp.mean(z, axis=0)
    d = z - m
    v = jnp.mean(d * d, axis=0)
    zn = d * lax.rsqrt(v + 1e-5) * pv_ref[1] + pv_ref[2]
    o_ref[...] = jnp.where(zn > 0, zn, pv_ref[3] * zn)


def _tc_post(aggp, x, w_root, pv):
    return pl.pallas_call(
        _tc_post_body,
        out_shape=jax.ShapeDtypeStruct((N, DH), _f32),
    )(aggp, x, w_root, pv)


def _tc_postpre_body(aggp_ref, x_ref, w_ref, pv_ref, wrel_ref, h_ref, o_ref):
    r = pl.program_id(0)

    @pl.when(r == 0)
    def _():
        _tc_post_body(aggp_ref, x_ref, w_ref, pv_ref, h_ref)

    hh = jnp.dot(h_ref[...], wrel_ref[0], preferred_element_type=_f32)
    o_ref[0, 0] = hh[:, :DF]
    o_ref[1, 0] = hh[:, DF:]


def _tc_postpre(aggp, x, w_root, pv, w_rel):
    return pl.pallas_call(
        _tc_postpre_body,
        grid=(R,),
        in_specs=[
            pl.BlockSpec((NC * N, DF), lambda r: (0, 0)),
            pl.BlockSpec((N, DIN), lambda r: (0, 0)),
            pl.BlockSpec((DIN, DH), lambda r: (0, 0)),
            pl.BlockSpec((4, DH), lambda r: (0, 0)),
            pl.BlockSpec((1, DIN, DH), lambda r: (r, 0, 0)),
        ],
        out_specs=[
            pl.BlockSpec((N, DH), lambda r: (0, 0)),
            pl.BlockSpec((NC, 1, N, DF), lambda r: (0, r, 0, 0)),
        ],
        out_shape=[
            jax.ShapeDtypeStruct((N, DH), _f32),
            jax.ShapeDtypeStruct((NC, R, N, DF), _f32),
        ],
    )(aggp, x, w_root, pv, w_rel)


def kernel(x, edge_index, edge_attr, params):
    src = edge_index[0]
    dst = edge_index[1]
    et = edge_attr[:, 4].astype(_i32)

    gidx, norm = _sc_prep(src, dst, et)

    hall = _tc_pre(x, params["W_rel1"])

    def pv(i):
        return jnp.stack([params[f"b{i}"], params[f"g{i}"],
                          params[f"be{i}"], params[f"a{i}"]])

    aggp = _sc_agg(hall.reshape(NC * RN, DF), gidx, dst, norm)
    h1, hall = _tc_postpre(aggp, x, params["W_root1"], pv("1"),
                           params["W_rel2"])
    aggp = _sc_agg(hall.reshape(NC * RN, DF), gidx, dst, norm)
    h2, hall = _tc_postpre(aggp, h1, params["W_root2"], pv("2"),
                           params["W_rel3"])
    aggp = _sc_agg(hall.reshape(NC * RN, DF), gidx, dst, norm)
    return _tc_post(aggp, h2, params["W_root3"], pv("3"))
